# Initial kernel scaffold; baseline (speedup 1.0000x reference)
import jax
import jax.numpy as jnp
from jax.experimental import pallas as pl


def kernel(xyz, pmt, mad, dim, nor, loc, params):
    raise NotImplementedError("write your pallas kernel here")


# speedup vs baseline: 18.0608x; 18.0608x over previous
import functools

import jax
import jax.numpy as jnp
from jax import lax
from jax.experimental import pallas as pl
from jax.experimental.pallas import tpu as pltpu
from jax.experimental.pallas import tpu_sc as plsc

BS = 8
N1 = 2048
M1 = 1024
M2 = 512
KNN = 32
NW = 32

_BIG = 1e30


@functools.lru_cache(maxsize=None)
def _sc_gather_fn(n_rows, n_idx, d, dtype_name):
    dtype = jnp.dtype(dtype_name)
    per_w = n_idx // NW
    assert n_idx % NW == 0 and per_w % 8 == 0 and d % 16 == 0
    ch = per_w
    while ch * d * 4 > 280 * 1024:
        ch //= 2
    n_chunks = per_w // ch
    mesh = plsc.VectorSubcoreMesh(core_axis_name="c", subcore_axis_name="s")

    @functools.partial(
        pl.kernel,
        out_type=jax.ShapeDtypeStruct((n_idx, d), dtype),
        mesh=mesh,
        compiler_params=pltpu.CompilerParams(use_tc_tiling_on_sc=False),
        scratch_types=[
            pltpu.VMEM((per_w,), jnp.int32),
            pltpu.VMEM((ch, d), dtype),
            pltpu.SemaphoreType.DMA,
        ],
    )
    def gather(table_hbm, idx_hbm, out_hbm, idx_v, buf, sem):
        wid = lax.axis_index("s") * 2 + lax.axis_index("c")
        base = wid * per_w
        pltpu.sync_copy(idx_hbm.at[pl.ds(base, per_w)], idx_v)
        for c in range(n_chunks):
            src = table_hbm.at[idx_v.at[pl.ds(c * ch, ch)]] if n_chunks > 1 \
                else table_hbm.at[idx_v]
            pltpu.async_copy(src, buf, sem).wait()
            pltpu.sync_copy(buf, out_hbm.at[pl.ds(base + c * ch, ch)])

    return gather


def _gather_rows(table, idx):
    fn = _sc_gather_fn(table.shape[0], idx.shape[0], table.shape[1],
                       table.dtype.name)
    return fn(table, idx)


def _tri_body(x_ref, w1, b1, w2, b2, wf1, bf1, wf2, bf2, out_ref):
    x = x_ref[...]
    h1 = jnp.maximum(jnp.dot(x, w1[...], preferred_element_type=jnp.float32)
                     + b1[...], 0.0)
    h2 = jnp.maximum(jnp.dot(h1, w2[...], preferred_element_type=jnp.float32)
                     + b2[...], 0.0)
    f1 = jnp.maximum(jnp.dot(h2, wf1[...], preferred_element_type=jnp.float32)
                     + bf1[...], 0.0)
    cst = jnp.maximum(jnp.dot(f1, wf2[...], preferred_element_type=jnp.float32)
                      + bf2[...], 0.0)
    pad = jnp.zeros((x.shape[0], 4), jnp.float32)
    out_ref[...] = jnp.concatenate(
        [x[:, 0:3], x[:, 8:11], x[:, 3:5], x[:, 3:7],
         h2[:, 64:96], h2[:, 32:64], h2[:, 160:192], cst, pad], axis=1)


def _tri_table(x18, w1, b1, w2, b2, wf1, bf1, wf2, bf2):
    n = x18.shape[0]
    pb = 2048
    full = lambda a: pl.BlockSpec(a.shape, lambda i: (0,) * a.ndim)
    return pl.pallas_call(
        _tri_body,
        grid=(n // pb,),
        in_specs=[pl.BlockSpec((pb, 18), lambda i: (i, 0)),
                  full(w1), full(b1), full(w2), full(b2),
                  full(wf1), full(bf1), full(wf2), full(bf2)],
        out_specs=pl.BlockSpec((pb, 144), lambda i: (i, 0)),
        out_shape=jax.ShapeDtypeStruct((n, 144), jnp.float32),
    )(x18, w1, b1, w2, b2, wf1, bf1, wf2, bf2)


def _knn_body(x_ref, yt_ref, out_ref, *, n, rows):
    b = pl.program_id(0)
    x = x_ref[0]
    yt = yt_ref[0]
    rsq = (x[:, 0:1] * x[:, 0:1] + x[:, 1:2] * x[:, 1:2]
           + x[:, 2:3] * x[:, 2:3])
    csq = yt[0:1] * yt[0:1] + yt[1:2] * yt[1:2] + yt[2:3] * yt[2:3]
    prod = (x[:, 0:1] * yt[0:1] + x[:, 1:2] * yt[1:2] + x[:, 2:3] * yt[2:3])
    d = rsq + csq - 2.0 * prod
    iota = lax.broadcasted_iota(jnp.int32, (rows, n), 1)
    cols = []
    for _ in range(KNN):
        mn = jnp.min(d, axis=1, keepdims=True)
        cidx = jnp.min(jnp.where(d == mn, iota, n), axis=1, keepdims=True)
        cols.append(cidx)
        d = jnp.where(iota == cidx, _BIG, d)
    out_ref[0] = jnp.concatenate(cols, axis=1) + b * n


def _knn(xyz, n):
    rows = 256
    xyzt = jnp.transpose(xyz, (0, 2, 1))
    body = functools.partial(_knn_body, n=n, rows=rows)
    return pl.pallas_call(
        body,
        grid=(BS, n // rows),
        in_specs=[pl.BlockSpec((1, rows, 3), lambda b, r: (b, r, 0)),
                  pl.BlockSpec((1, 3, n), lambda b, r: (b, 0, 0))],
        out_specs=pl.BlockSpec((1, rows, KNN), lambda b, r: (b, r, 0)),
        out_shape=jax.ShapeDtypeStruct((BS, n, KNN), jnp.int32),
    )(xyz, xyzt)


def _fps_body(x_ref, y_ref, z_ref, out_ref, *, n, m, s, sm):
    X, Y, Z = x_ref[...], y_ref[...], z_ref[...]
    iota_p = (lax.broadcasted_iota(jnp.int32, (BS, s, 128), 1) * 128
              + lax.broadcasted_iota(jnp.int32, (BS, s, 128), 2))
    iota_o = (lax.broadcasted_iota(jnp.int32, (BS, sm, 128), 1) * 128
              + lax.broadcasted_iota(jnp.int32, (BS, sm, 128), 2))

    def body(i, state):
        dists, last, acc = state
        sel = iota_p == last
        zero = jnp.float32(0.0)
        lx = jnp.sum(jnp.where(sel, X, zero), axis=(1, 2), keepdims=True)
        ly = jnp.sum(jnp.where(sel, Y, zero), axis=(1, 2), keepdims=True)
        lz = jnp.sum(jnp.where(sel, Z, zero), axis=(1, 2), keepdims=True)
        dx, dy, dz = X - lx, Y - ly, Z - lz
        d = dx * dx + dy * dy + dz * dz
        dists = jnp.minimum(dists, d)
        mx = jnp.max(dists, axis=(1, 2), keepdims=True)
        nxt = jnp.min(jnp.where(dists == mx, iota_p, n), axis=(1, 2),
                      keepdims=True)
        acc = jnp.where(iota_o == i, nxt, acc)
        return dists, nxt, acc

    init = (jnp.full((BS, s, 128), 1e10, jnp.float32),
            jnp.zeros((BS, 1, 1), jnp.int32),
            jnp.zeros((BS, sm, 128), jnp.int32))
    _, _, acc = lax.fori_loop(1, m, body, init)
    boffs = lax.broadcasted_iota(jnp.int32, (BS, sm, 128), 0) * n
    out_ref[...] = acc + boffs


def _fps(xflat, yflat, zflat, n, m):
    s, sm = n // 128, m // 128
    body = functools.partial(_fps_body, n=n, m=m, s=s, sm=sm)
    out = pl.pallas_call(
        body,
        out_shape=jax.ShapeDtypeStruct((BS, sm, 128), jnp.int32),
    )(xflat.reshape(BS, s, 128), yflat.reshape(BS, s, 128),
      zflat.reshape(BS, s, 128))
    return out.reshape(BS * m)


def _attn_body(g_ref, c_ref, *refs, mb, k, co, specs, out_w):
    nb = len(specs)
    wrefs, out_ref = refs[:-1], refs[-1]
    g = g_ref[...]
    cen = c_ref[...]
    scale = 1.0 / (co ** 0.5)
    feats = []
    for i, (qc, fc, rc, crc, sign) in enumerate(specs):
        wq, bq, wk, wv, bv = (wrefs[i * 5 + j][...] for j in range(5))
        q = jnp.dot(cen[:, qc[0]:qc[1]], wq,
                    preferred_element_type=jnp.float32) + bq
        gcat = jnp.concatenate([g[:, fc[0]:fc[1]], g[:, rc[0]:rc[1]]], axis=1)
        cg = gcat.shape[1]
        pk = jnp.dot(gcat, wk[:cg], preferred_element_type=jnp.float32)
        pv = jnp.dot(gcat, wv[:cg], preferred_element_type=jnp.float32)
        craw = cen[:, crc[0]:crc[1]]
        nr = crc[1] - crc[0]
        wvr = wv[cg - nr:cg] if sign < 0 else wv[cg:cg + nr]
        cv = bv + sign * jnp.dot(craw, wvr,
                                 preferred_element_type=jnp.float32)
        pk3 = pk.reshape(mb, k, co)
        pv3 = pv.reshape(mb, k, co)
        q3 = q.reshape(mb, 1, co)
        s3 = jnp.sum(q3 * pk3, axis=2, keepdims=True) * scale
        mx = jnp.max(s3, axis=1, keepdims=True)
        e3 = jnp.exp(s3 - mx)
        a3 = e3 / jnp.sum(e3, axis=1, keepdims=True)
        o = jnp.sum(a3 * pv3, axis=1) + cv
        feats.append(o)
    faw, fab = wrefs[nb * 5][...], wrefs[nb * 5 + 1][...]
    ss = [jnp.sum(f * faw, axis=1, keepdims=True) + fab for f in feats]
    mx = jnp.maximum(jnp.maximum(ss[0], ss[1]), jnp.maximum(ss[2], ss[3]))
    es = [jnp.exp(x - mx) for x in ss]
    z = es[0] + es[1] + es[2] + es[3]
    fused = sum((es[i] / z) * feats[i] for i in range(nb))
    outs = [cen[:, 0:12]] + [f + fused for f in feats]
    pad = out_w - 12 - nb * co
    if pad:
        outs.append(jnp.zeros((mb, pad), jnp.float32))
    out_ref[...] = jnp.concatenate(outs, axis=1)


def _attention(gathered, centers, specs, weights, co, out_w):
    mt, c = centers.shape
    mb = 256
    k = KNN
    flat_w = []
    for (wq, bq, wk, wv, bv) in weights[:-1]:
        flat_w += [wq, bq, wk, wv, bv]
    flat_w += list(weights[-1])
    body = functools.partial(_attn_body, mb=mb, k=k, co=co, specs=specs,
                             out_w=out_w)
    full = lambda a: pl.BlockSpec(a.shape, lambda i: (0,) * a.ndim)
    return pl.pallas_call(
        body,
        grid=(mt // mb,),
        in_specs=[pl.BlockSpec((mb * k, c), lambda i: (i, 0)),
                  pl.BlockSpec((mb, c), lambda i: (i, 0))]
        + [full(a) for a in flat_w],
        out_specs=pl.BlockSpec((mb, out_w), lambda i: (i, 0)),
        out_shape=jax.ShapeDtypeStruct((mt, out_w), jnp.float32),
    )(gathered, centers, *flat_w)


def _final_body(t_ref, *refs, co):
    wrefs, out_ref = refs[:-1], refs[-1]
    t = t_ref[...]
    segs = [(12, 140, [(3, 6)]), (140, 268, [(6, 8), (6, 8)]),
            (268, 396, [(8, 12), (8, 12)]), (396, 524, [(0, 3)])]
    gs = []
    for bi, (f0, f1, extras) in enumerate(segs):
        rows = []
        for b in range(BS):
            blk = t[b * M2:(b + 1) * M2]
            cols = [blk[:, f0:f1]] + [blk[:, a:bb] for (a, bb) in extras]
            cat = jnp.concatenate(cols, axis=1)
            rows.append(jnp.max(cat, axis=0, keepdims=True))
        g = jnp.concatenate(rows, axis=0)
        w1, b1, w2, b2 = (wrefs[bi * 4 + j][...] for j in range(4))
        h = jnp.maximum(jnp.dot(g, w1, preferred_element_type=jnp.float32)
                        + b1, 0.0)
        gs.append(jnp.maximum(
            jnp.dot(h, w2, preferred_element_type=jnp.float32) + b2, 0.0))
    faw, fab = wrefs[16][...], wrefs[17][...]
    ss = [jnp.sum(f * faw, axis=1, keepdims=True) + fab for f in gs]
    mx = jnp.maximum(jnp.maximum(ss[0], ss[1]), jnp.maximum(ss[2], ss[3]))
    es = [jnp.exp(x - mx) for x in ss]
    z = es[0] + es[1] + es[2] + es[3]
    fused = sum((es[i] / z) * gs[i] for i in range(4))
    gs = [f + fused for f in gs]
    favg = (gs[0] + gs[1] + gs[2] + gs[3]) * 0.25
    hw1, hb1, hw2, hb2 = (wrefs[18 + j][...] for j in range(4))
    h = jnp.maximum(jnp.dot(favg, hw1, preferred_element_type=jnp.float32)
                    + hb1, 0.0)
    out_ref[...] = jnp.dot(h, hw2, preferred_element_type=jnp.float32) + hb2


def _final(table3, wlist):
    return pl.pallas_call(
        functools.partial(_final_body, co=256),
        out_shape=jax.ShapeDtypeStruct((BS, 50), jnp.float32),
    )(table3, *wlist)


def _r2(b):
    return b.reshape(1, -1)


def kernel(xyz, pmt, mad, dim, nor, loc, params):
    p = params
    tri = p["tri"]
    names = ["xyz", "pmt", "mad", "dim", "nor", "loc"]
    ins = [3, 5, 3, 1, 3, 3]
    offs = [0, 3, 8, 11, 12, 15]
    w1 = jnp.zeros((18, 96), jnp.float32)
    w2 = jnp.zeros((96, 192), jnp.float32)
    for i, nm in enumerate(names):
        w1 = w1.at[offs[i]:offs[i] + ins[i], 16 * i:16 * (i + 1)].set(
            tri[nm][0]["W"])
        w2 = w2.at[16 * i:16 * (i + 1), 32 * i:32 * (i + 1)].set(tri[nm][1]["W"])
    b1 = jnp.concatenate([tri[nm][0]["b"] for nm in names]).reshape(1, 96)
    b2 = jnp.concatenate([tri[nm][1]["b"] for nm in names]).reshape(1, 192)
    x18 = jnp.concatenate(
        [xyz, pmt, mad, dim[..., None], nor, loc], axis=-1).reshape(-1, 18)
    table1 = _tri_table(x18, w1, b1, w2, b2,
                        tri["fea"][0]["W"], _r2(tri["fea"][0]["b"]),
                        tri["fea"][1]["W"], _r2(tri["fea"][1]["b"]))

    idx1 = _knn(xyz, N1)
    fps1 = _fps(xyz[:, :, 0], xyz[:, :, 1], xyz[:, :, 2], N1, M1)
    nbr_idx1 = _gather_rows(idx1.reshape(BS * N1, KNN), fps1)
    g1 = _gather_rows(table1, nbr_idx1.reshape(-1))
    c1 = _gather_rows(table1, fps1)

    def attw(ap):
        return (ap["q"]["W"], _r2(ap["q"]["b"]), ap["k"]["W"], ap["v"]["W"],
                _r2(ap["v"]["b"]))

    specs1 = [((12, 44), (12, 44), (3, 6), (3, 6), -1.0),
              ((44, 76), (44, 76), (6, 8), (6, 8), 1.0),
              ((76, 108), (76, 108), (8, 12), (8, 12), 1.0),
              ((108, 140), (108, 140), (0, 3), (0, 3), -1.0)]
    w_1 = [attw(p["ssa1"]["attn_mad"]), attw(p["ssa1"]["attn_adj"]),
           attw(p["ssa1"]["attn_pt"]), attw(p["ssa1"]["attn_cst"]),
           (p["fa1"]["W"].reshape(1, 64), p["fa1"]["b"].reshape(1, 1))]
    table2 = _attention(g1, c1, specs1, w_1, 64, 272)

    cx = c1[:, 0].reshape(BS, M1)
    cy = c1[:, 1].reshape(BS, M1)
    cz = c1[:, 2].reshape(BS, M1)
    cxyz = c1[:, 0:3].reshape(BS, M1, 3)
    idx2 = _knn(cxyz, M1)
    fps2 = _fps(cx, cy, cz, M1, M2)
    nbr_idx2 = _gather_rows(idx2.reshape(BS * M1, KNN), fps2)
    g2 = _gather_rows(table2, nbr_idx2.reshape(-1))
    c2 = _gather_rows(table2, fps2)

    specs2 = [((12, 76), (12, 76), (3, 6), (3, 6), -1.0),
              ((76, 140), (76, 140), (6, 8), (6, 8), 1.0),
              ((140, 204), (140, 204), (8, 12), (8, 12), 1.0),
              ((204, 268), (204, 268), (0, 3), (0, 3), -1.0)]
    w_2 = [attw(p["ssa2"]["attn_mad"]), attw(p["ssa2"]["attn_adj"]),
           attw(p["ssa2"]["attn_pt"]), attw(p["ssa2"]["attn_cst"]),
           (p["fa2"]["W"].reshape(1, 128), p["fa2"]["b"].reshape(1, 1))]
    table3 = _attention(g2, c2, specs2, w_2, 128, 528)

    wlist = []
    for nm in ["mlp_mad", "mlp_adj", "mlp_pt", "mlp_cst"]:
        ml = p["ssa3"][nm]
        wlist += [ml[0]["W"], _r2(ml[0]["b"]), ml[1]["W"], _r2(ml[1]["b"])]
    wlist += [p["fa3"]["W"].reshape(1, 256), p["fa3"]["b"].reshape(1, 1)]
    wlist += [p["head"][0]["W"], _r2(p["head"][0]["b"]),
              p["head"][1]["W"], _r2(p["head"][1]["b"])]
    return _final(table3, wlist)


# speedup vs baseline: 26.9327x; 1.4912x over previous
import functools

import jax
import jax.numpy as jnp
from jax import lax
from jax.experimental import pallas as pl
from jax.experimental.pallas import tpu as pltpu
from jax.experimental.pallas import tpu_sc as plsc

BS = 8
N1 = 2048
M1 = 1024
M2 = 512
KNN = 32
NW = 32

_BIG = 1e30


@functools.lru_cache(maxsize=None)
def _sc_gather_fn(n_rows, n_idx, d, dtype_name, tiled):
    dtype = jnp.dtype(dtype_name)
    per_w = n_idx // NW
    assert n_idx % NW == 0 and per_w % 8 == 0 and d % 16 == 0
    ch = per_w
    while ch * d * 4 > 280 * 1024:
        ch //= 2
    n_chunks = per_w // ch
    mesh = plsc.VectorSubcoreMesh(core_axis_name="c", subcore_axis_name="s")

    @functools.partial(
        pl.kernel,
        out_type=jax.ShapeDtypeStruct((n_idx, d), dtype),
        mesh=mesh,
        compiler_params=pltpu.CompilerParams(use_tc_tiling_on_sc=tiled),
        scratch_types=[
            pltpu.VMEM((per_w,), jnp.int32),
            pltpu.VMEM((ch, d), dtype),
            pltpu.SemaphoreType.DMA,
        ],
    )
    def gather(table_hbm, idx_hbm, out_hbm, idx_v, buf, sem):
        wid = lax.axis_index("s") * 2 + lax.axis_index("c")
        base = wid * per_w
        pltpu.sync_copy(idx_hbm.at[pl.ds(base, per_w)], idx_v)
        for c in range(n_chunks):
            src = table_hbm.at[idx_v.at[pl.ds(c * ch, ch)]] if n_chunks > 1 \
                else table_hbm.at[idx_v]
            pltpu.async_copy(src, buf, sem).wait()
            pltpu.sync_copy(buf, out_hbm.at[pl.ds(base + c * ch, ch)])

    return gather


def _gather_rows(table, idx, tiled):
    fn = _sc_gather_fn(table.shape[0], idx.shape[0], table.shape[1],
                       table.dtype.name, tiled)
    return fn(table, idx)


def _tri_body(x_ref, w1, b1, w2, b2, wf1, bf1, wf2, bf2, f_ref, r_ref):
    x = x_ref[...]
    h1 = jnp.maximum(jnp.dot(x, w1[...], preferred_element_type=jnp.float32)
                     + b1[...], 0.0)
    h2 = jnp.maximum(jnp.dot(h1, w2[...], preferred_element_type=jnp.float32)
                     + b2[...], 0.0)
    f1 = jnp.maximum(jnp.dot(h2, wf1[...], preferred_element_type=jnp.float32)
                     + bf1[...], 0.0)
    cst = jnp.maximum(jnp.dot(f1, wf2[...], preferred_element_type=jnp.float32)
                      + bf2[...], 0.0)
    f_ref[...] = jnp.concatenate(
        [h2[:, 64:96], h2[:, 32:64], h2[:, 160:192], cst], axis=1)
    pad = jnp.zeros((x.shape[0], 4), jnp.float32)
    r_ref[...] = jnp.concatenate(
        [x[:, 0:3], x[:, 8:11], x[:, 3:5], x[:, 3:7], pad], axis=1)


def _tri_table(x18, w1, b1, w2, b2, wf1, bf1, wf2, bf2):
    n = x18.shape[0]
    pb = 2048
    full = lambda a: pl.BlockSpec(a.shape, lambda i: (0,) * a.ndim)
    return pl.pallas_call(
        _tri_body,
        grid=(n // pb,),
        in_specs=[pl.BlockSpec((pb, 18), lambda i: (i, 0)),
                  full(w1), full(b1), full(w2), full(b2),
                  full(wf1), full(bf1), full(wf2), full(bf2)],
        out_specs=[pl.BlockSpec((pb, 128), lambda i: (i, 0)),
                   pl.BlockSpec((pb, 16), lambda i: (i, 0))],
        out_shape=[jax.ShapeDtypeStruct((n, 128), jnp.float32),
                   jax.ShapeDtypeStruct((n, 16), jnp.float32)],
    )(x18, w1, b1, w2, b2, wf1, bf1, wf2, bf2)


def _knn_body(x_ref, yt_ref, out_ref, *, n, rows):
    b = pl.program_id(0)
    x = x_ref[0]
    yt = yt_ref[0]
    rsq = (x[:, 0:1] * x[:, 0:1] + x[:, 1:2] * x[:, 1:2]
           + x[:, 2:3] * x[:, 2:3])
    csq = yt[0:1] * yt[0:1] + yt[1:2] * yt[1:2] + yt[2:3] * yt[2:3]
    prod = (x[:, 0:1] * yt[0:1] + x[:, 1:2] * yt[1:2] + x[:, 2:3] * yt[2:3])
    d = rsq + csq - 2.0 * prod
    iota = lax.broadcasted_iota(jnp.int32, (rows, n), 1)
    cols = []
    for _ in range(KNN):
        mn = jnp.min(d, axis=1, keepdims=True)
        t = jnp.where(d == mn, iota, n)
        cidx = jnp.min(t, axis=1, keepdims=True)
        cols.append(cidx)
        d = jnp.where(t == cidx, _BIG, d)
    out_ref[0] = jnp.concatenate(cols, axis=1) + b * n


def _knn(cxyz, xyzt, n, m):
    rows = 256
    body = functools.partial(_knn_body, n=n, rows=rows)
    return pl.pallas_call(
        body,
        grid=(BS, m // rows),
        in_specs=[pl.BlockSpec((1, rows, 3), lambda b, r: (b, r, 0)),
                  pl.BlockSpec((1, 3, n), lambda b, r: (b, 0, 0))],
        out_specs=pl.BlockSpec((1, rows, KNN), lambda b, r: (b, r, 0)),
        out_shape=jax.ShapeDtypeStruct((BS, m, KNN), jnp.int32),
    )(cxyz, xyzt)


def _fps_body(x_ref, y_ref, z_ref, out_ref, *, n, m, s, sm):
    X, Y, Z = x_ref[...], y_ref[...], z_ref[...]
    iota_p = (lax.broadcasted_iota(jnp.int32, (BS, s, 128), 1) * 128
              + lax.broadcasted_iota(jnp.int32, (BS, s, 128), 2))
    iota_o = (lax.broadcasted_iota(jnp.int32, (BS, sm, 128), 1) * 128
              + lax.broadcasted_iota(jnp.int32, (BS, sm, 128), 2))

    def body(i, state):
        dists, last, acc = state
        sel = iota_p == last
        zero = jnp.float32(0.0)
        lx = jnp.sum(jnp.where(sel, X, zero), axis=(1, 2), keepdims=True)
        ly = jnp.sum(jnp.where(sel, Y, zero), axis=(1, 2), keepdims=True)
        lz = jnp.sum(jnp.where(sel, Z, zero), axis=(1, 2), keepdims=True)
        dx, dy, dz = X - lx, Y - ly, Z - lz
        d = dx * dx + dy * dy + dz * dz
        dists = jnp.minimum(dists, d)
        mx = jnp.max(dists, axis=(1, 2), keepdims=True)
        nxt = jnp.min(jnp.where(dists == mx, iota_p, n), axis=(1, 2),
                      keepdims=True)
        acc = jnp.where(iota_o == i, nxt, acc)
        return dists, nxt, acc

    init = (jnp.full((BS, s, 128), 1e10, jnp.float32),
            jnp.zeros((BS, 1, 1), jnp.int32),
            jnp.zeros((BS, sm, 128), jnp.int32))
    _, _, acc = lax.fori_loop(1, m, body, init)
    boffs = lax.broadcasted_iota(jnp.int32, (BS, sm, 128), 0) * n
    out_ref[...] = acc + boffs


def _fps(xflat, yflat, zflat, n, m):
    s, sm = n // 128, m // 128
    body = functools.partial(_fps_body, n=n, m=m, s=s, sm=sm)
    out = pl.pallas_call(
        body,
        out_shape=jax.ShapeDtypeStruct((BS, sm, 128), jnp.int32),
    )(xflat.reshape(BS, s, 128), yflat.reshape(BS, s, 128),
      zflat.reshape(BS, s, 128))
    return out.reshape(BS * m)


def _attn_body(gf_ref, gr_ref, cf_ref, cr_ref, *refs, mb, k, co, specs):
    nb = len(specs)
    wrefs, out_ref = refs[:-1], refs[-1]
    gf = gf_ref[...]
    gr = gr_ref[...]
    cf = cf_ref[...]
    cr = cr_ref[...]
    scale = 1.0 / (co ** 0.5)
    feats = []
    for i, (fc, rc, sign) in enumerate(specs):
        wq, bq, wk, wv, bv = (wrefs[i * 5 + j][...] for j in range(5))
        q = jnp.dot(cf[:, fc[0]:fc[1]], wq,
                    preferred_element_type=jnp.float32) + bq
        gcat = jnp.concatenate([gf[:, fc[0]:fc[1]], gr[:, rc[0]:rc[1]]],
                               axis=1)
        cg = gcat.shape[1]
        pk = jnp.dot(gcat, wk[:cg], preferred_element_type=jnp.float32)
        pv = jnp.dot(gcat, wv[:cg], preferred_element_type=jnp.float32)
        craw = cr[:, rc[0]:rc[1]]
        nr = rc[1] - rc[0]
        wvr = wv[cg - nr:cg] if sign < 0 else wv[cg:cg + nr]
        cv = bv + sign * jnp.dot(craw, wvr,
                                 preferred_element_type=jnp.float32)
        pk3 = pk.reshape(mb, k, co)
        pv3 = pv.reshape(mb, k, co)
        q3 = q.reshape(mb, 1, co)
        s3 = jnp.sum(q3 * pk3, axis=2, keepdims=True) * scale
        mx = jnp.max(s3, axis=1, keepdims=True)
        e3 = jnp.exp(s3 - mx)
        a3 = e3 / jnp.sum(e3, axis=1, keepdims=True)
        o = jnp.sum(a3 * pv3, axis=1) + cv
        feats.append(o)
    faw, fab = wrefs[nb * 5][...], wrefs[nb * 5 + 1][...]
    ss = [jnp.sum(f * faw, axis=1, keepdims=True) + fab for f in feats]
    mx = jnp.maximum(jnp.maximum(ss[0], ss[1]), jnp.maximum(ss[2], ss[3]))
    es = [jnp.exp(x - mx) for x in ss]
    z = es[0] + es[1] + es[2] + es[3]
    fused = sum((es[i] / z) * feats[i] for i in range(nb))
    out_ref[...] = jnp.concatenate([f + fused for f in feats], axis=1)


def _attention(gfeat, graw, cfeat, craw, specs, weights, co):
    mt, c = cfeat.shape
    mb = 256
    k = KNN
    flat_w = []
    for (wq, bq, wk, wv, bv) in weights[:-1]:
        flat_w += [wq, bq, wk, wv, bv]
    flat_w += list(weights[-1])
    body = functools.partial(_attn_body, mb=mb, k=k, co=co, specs=specs)
    full = lambda a: pl.BlockSpec(a.shape, lambda i: (0,) * a.ndim)
    return pl.pallas_call(
        body,
        grid=(mt // mb,),
        in_specs=[pl.BlockSpec((mb * k, c), lambda i: (i, 0)),
                  pl.BlockSpec((mb * k, 16), lambda i: (i, 0)),
                  pl.BlockSpec((mb, c), lambda i: (i, 0)),
                  pl.BlockSpec((mb, 16), lambda i: (i, 0))]
        + [full(a) for a in flat_w],
        out_specs=pl.BlockSpec((mb, 4 * co), lambda i: (i, 0)),
        out_shape=jax.ShapeDtypeStruct((mt, 4 * co), jnp.float32),
    )(gfeat, graw, cfeat, craw, *flat_w)


def _final_body(t_ref, r_ref, *refs):
    wrefs, out_ref = refs[:-1], refs[-1]
    t = t_ref[...]
    r = r_ref[...]
    segs = [((0, 128), [(3, 6)]), ((128, 256), [(6, 8), (6, 8)]),
            ((256, 384), [(8, 12), (8, 12)]), ((384, 512), [(0, 3)])]
    gs = []
    for bi, ((f0, f1), extras) in enumerate(segs):
        rows = []
        for b in range(BS):
            tb = t[b * M2:(b + 1) * M2]
            rb = r[b * M2:(b + 1) * M2]
            cols = [tb[:, f0:f1]] + [rb[:, a:bb] for (a, bb) in extras]
            cat = jnp.concatenate(cols, axis=1)
            rows.append(jnp.max(cat, axis=0, keepdims=True))
        g = jnp.concatenate(rows, axis=0)
        w1, b1, w2, b2 = (wrefs[bi * 4 + j][...] for j in range(4))
        h = jnp.maximum(jnp.dot(g, w1, preferred_element_type=jnp.float32)
                        + b1, 0.0)
        gs.append(jnp.maximum(
            jnp.dot(h, w2, preferred_element_type=jnp.float32) + b2, 0.0))
    faw, fab = wrefs[16][...], wrefs[17][...]
    ss = [jnp.sum(f * faw, axis=1, keepdims=True) + fab for f in gs]
    mx = jnp.maximum(jnp.maximum(ss[0], ss[1]), jnp.maximum(ss[2], ss[3]))
    es = [jnp.exp(x - mx) for x in ss]
    z = es[0] + es[1] + es[2] + es[3]
    fused = sum((es[i] / z) * gs[i] for i in range(4))
    gs = [f + fused for f in gs]
    favg = (gs[0] + gs[1] + gs[2] + gs[3]) * 0.25
    hw1, hb1, hw2, hb2 = (wrefs[18 + j][...] for j in range(4))
    h = jnp.maximum(jnp.dot(favg, hw1, preferred_element_type=jnp.float32)
                    + hb1, 0.0)
    out_ref[...] = jnp.dot(h, hw2, preferred_element_type=jnp.float32) + hb2


def _final(table3, craw, wlist):
    return pl.pallas_call(
        _final_body,
        out_shape=jax.ShapeDtypeStruct((BS, 50), jnp.float32),
    )(table3, craw, *wlist)


def _r2(b):
    return b.reshape(1, -1)


def kernel(xyz, pmt, mad, dim, nor, loc, params):
    p = params
    tri = p["tri"]
    names = ["xyz", "pmt", "mad", "dim", "nor", "loc"]
    ins = [3, 5, 3, 1, 3, 3]
    offs = [0, 3, 8, 11, 12, 15]
    w1 = jnp.zeros((18, 96), jnp.float32)
    w2 = jnp.zeros((96, 192), jnp.float32)
    for i, nm in enumerate(names):
        w1 = w1.at[offs[i]:offs[i] + ins[i], 16 * i:16 * (i + 1)].set(
            tri[nm][0]["W"])
        w2 = w2.at[16 * i:16 * (i + 1), 32 * i:32 * (i + 1)].set(tri[nm][1]["W"])
    b1 = jnp.concatenate([tri[nm][0]["b"] for nm in names]).reshape(1, 96)
    b2 = jnp.concatenate([tri[nm][1]["b"] for nm in names]).reshape(1, 192)
    x18 = jnp.concatenate(
        [xyz, pmt, mad, dim[..., None], nor, loc], axis=-1).reshape(-1, 18)
    t1f, t1r = _tri_table(x18, w1, b1, w2, b2,
                          tri["fea"][0]["W"], _r2(tri["fea"][0]["b"]),
                          tri["fea"][1]["W"], _r2(tri["fea"][1]["b"]))

    def attw(ap):
        return (ap["q"]["W"], _r2(ap["q"]["b"]), ap["k"]["W"], ap["v"]["W"],
                _r2(ap["v"]["b"]))

    fps1 = _fps(xyz[:, :, 0], xyz[:, :, 1], xyz[:, :, 2], N1, M1)
    c1f = _gather_rows(t1f, fps1, True)
    c1r = _gather_rows(t1r, fps1, False)
    xyzt = jnp.transpose(xyz, (0, 2, 1))
    cxyz1 = c1r[:, 0:3].reshape(BS, M1, 3)
    nbr1 = _knn(cxyz1, xyzt, N1, M1).reshape(-1)
    g1f = _gather_rows(t1f, nbr1, True)
    g1r = _gather_rows(t1r, nbr1, False)

    specs1 = [((0, 32), (3, 6), -1.0), ((32, 64), (6, 8), 1.0),
              ((64, 96), (8, 12), 1.0), ((96, 128), (0, 3), -1.0)]
    w_1 = [attw(p["ssa1"]["attn_mad"]), attw(p["ssa1"]["attn_adj"]),
           attw(p["ssa1"]["attn_pt"]), attw(p["ssa1"]["attn_cst"]),
           (p["fa1"]["W"].reshape(1, 64), p["fa1"]["b"].reshape(1, 1))]
    t2f = _attention(g1f, g1r, c1f, c1r, specs1, w_1, 64)

    cx = c1r[:, 0].reshape(BS, M1)
    cy = c1r[:, 1].reshape(BS, M1)
    cz = c1r[:, 2].reshape(BS, M1)
    fps2 = _fps(cx, cy, cz, M1, M2)
    c2f = _gather_rows(t2f, fps2, True)
    c2r = _gather_rows(c1r, fps2, False)
    cxyzt = jnp.transpose(cxyz1, (0, 2, 1))
    cxyz2 = c2r[:, 0:3].reshape(BS, M2, 3)
    nbr2 = _knn(cxyz2, cxyzt, M1, M2).reshape(-1)
    g2f = _gather_rows(t2f, nbr2, True)
    g2r = _gather_rows(c1r, nbr2, False)

    specs2 = [((0, 64), (3, 6), -1.0), ((64, 128), (6, 8), 1.0),
              ((128, 192), (8, 12), 1.0), ((192, 256), (0, 3), -1.0)]
    w_2 = [attw(p["ssa2"]["attn_mad"]), attw(p["ssa2"]["attn_adj"]),
           attw(p["ssa2"]["attn_pt"]), attw(p["ssa2"]["attn_cst"]),
           (p["fa2"]["W"].reshape(1, 128), p["fa2"]["b"].reshape(1, 1))]
    t3f = _attention(g2f, g2r, c2f, c2r, specs2, w_2, 128)

    wlist = []
    for nm in ["mlp_mad", "mlp_adj", "mlp_pt", "mlp_cst"]:
        ml = p["ssa3"][nm]
        wlist += [ml[0]["W"], _r2(ml[0]["b"]), ml[1]["W"], _r2(ml[1]["b"])]
    wlist += [p["fa3"]["W"].reshape(1, 256), p["fa3"]["b"].reshape(1, 1)]
    wlist += [p["head"][0]["W"], _r2(p["head"][0]["b"]),
              p["head"][1]["W"], _r2(p["head"][1]["b"])]
    return _final(t3f, c2r, wlist)


# speedup vs baseline: 27.4037x; 1.0175x over previous
import functools

import jax
import jax.numpy as jnp
from jax import lax
from jax.experimental import pallas as pl
from jax.experimental.pallas import tpu as pltpu
from jax.experimental.pallas import tpu_sc as plsc

BS = 8
N1 = 2048
M1 = 1024
M2 = 512
KNN = 32
NW = 32

_BIG = 1e30


@functools.lru_cache(maxsize=None)
def _sc_gather_fn(n_rows, n_idx, d, dtype_name, tiled):
    dtype = jnp.dtype(dtype_name)
    per_w = n_idx // NW
    assert n_idx % NW == 0 and per_w % 8 == 0 and d % 16 == 0
    ch = per_w
    while ch * d * 4 > 150 * 1024:
        ch //= 2
    n_chunks = per_w // ch
    mesh = plsc.VectorSubcoreMesh(core_axis_name="c", subcore_axis_name="s")

    @functools.partial(
        pl.kernel,
        out_type=jax.ShapeDtypeStruct((n_idx, d), dtype),
        mesh=mesh,
        compiler_params=pltpu.CompilerParams(use_tc_tiling_on_sc=tiled),
        scratch_types=[
            pltpu.VMEM((per_w,), jnp.int32),
            pltpu.VMEM((ch, d), dtype),
            pltpu.VMEM((ch, d), dtype),
            pltpu.SemaphoreType.DMA,
            pltpu.SemaphoreType.DMA,
            pltpu.SemaphoreType.DMA,
        ],
    )
    def gather(table_hbm, idx_hbm, out_hbm, idx_v, buf0, buf1, gsem, ws0, ws1):
        wid = lax.axis_index("s") * 2 + lax.axis_index("c")
        base = wid * per_w
        pltpu.sync_copy(idx_hbm.at[pl.ds(base, per_w)], idx_v)
        bufs, wsems = (buf0, buf1), (ws0, ws1)
        writes = [None, None]
        for c in range(n_chunks):
            b = c % 2
            if writes[b] is not None:
                writes[b].wait()
            src = table_hbm.at[idx_v.at[pl.ds(c * ch, ch)]] if n_chunks > 1 \
                else table_hbm.at[idx_v]
            pltpu.async_copy(src, bufs[b], gsem).wait()
            writes[b] = pltpu.async_copy(
                bufs[b], out_hbm.at[pl.ds(base + c * ch, ch)], wsems[b])
        for w in writes:
            if w is not None:
                w.wait()

    return gather


def _gather_rows(table, idx, tiled):
    fn = _sc_gather_fn(table.shape[0], idx.shape[0], table.shape[1],
                       table.dtype.name, tiled)
    return fn(table, idx)


def _tri_body(x_ref, w1, b1, w2, b2, wf1, bf1, wf2, bf2, f_ref, r_ref):
    x = x_ref[...]
    h1 = jnp.maximum(jnp.dot(x, w1[...], preferred_element_type=jnp.float32)
                     + b1[...], 0.0)
    h2 = jnp.maximum(jnp.dot(h1, w2[...], preferred_element_type=jnp.float32)
                     + b2[...], 0.0)
    f1 = jnp.maximum(jnp.dot(h2, wf1[...], preferred_element_type=jnp.float32)
                     + bf1[...], 0.0)
    cst = jnp.maximum(jnp.dot(f1, wf2[...], preferred_element_type=jnp.float32)
                      + bf2[...], 0.0)
    f_ref[...] = jnp.concatenate(
        [h2[:, 64:96], h2[:, 32:64], h2[:, 160:192], cst], axis=1)
    pad = jnp.zeros((x.shape[0], 4), jnp.float32)
    r_ref[...] = jnp.concatenate(
        [x[:, 0:3], x[:, 8:11], x[:, 3:5], x[:, 3:7], pad], axis=1)


def _tri_table(x18, w1, b1, w2, b2, wf1, bf1, wf2, bf2):
    n = x18.shape[0]
    pb = 2048
    full = lambda a: pl.BlockSpec(a.shape, lambda i: (0,) * a.ndim)
    return pl.pallas_call(
        _tri_body,
        grid=(n // pb,),
        in_specs=[pl.BlockSpec((pb, 18), lambda i: (i, 0)),
                  full(w1), full(b1), full(w2), full(b2),
                  full(wf1), full(bf1), full(wf2), full(bf2)],
        out_specs=[pl.BlockSpec((pb, 128), lambda i: (i, 0)),
                   pl.BlockSpec((pb, 16), lambda i: (i, 0))],
        out_shape=[jax.ShapeDtypeStruct((n, 128), jnp.float32),
                   jax.ShapeDtypeStruct((n, 16), jnp.float32)],
    )(x18, w1, b1, w2, b2, wf1, bf1, wf2, bf2)


def _knn_body(x_ref, yt_ref, out_ref, *, n, rows):
    b = pl.program_id(0)
    x = x_ref[0]
    yt = yt_ref[0]
    rsq = (x[:, 0:1] * x[:, 0:1] + x[:, 1:2] * x[:, 1:2]
           + x[:, 2:3] * x[:, 2:3])
    csq = yt[0:1] * yt[0:1] + yt[1:2] * yt[1:2] + yt[2:3] * yt[2:3]
    prod = (x[:, 0:1] * yt[0:1] + x[:, 1:2] * yt[1:2] + x[:, 2:3] * yt[2:3])
    d = rsq + csq - 2.0 * prod
    iota = lax.broadcasted_iota(jnp.int32, (rows, n), 1)
    cols = []
    for _ in range(KNN):
        mn = jnp.min(d, axis=1, keepdims=True)
        t = jnp.where(d == mn, iota, n)
        cidx = jnp.min(t, axis=1, keepdims=True)
        cols.append(cidx)
        d = jnp.where(t == cidx, _BIG, d)
    out_ref[0] = jnp.concatenate(cols, axis=1) + b * n


def _knn(cxyz, xyzt, n, m):
    rows = 256
    body = functools.partial(_knn_body, n=n, rows=rows)
    return pl.pallas_call(
        body,
        grid=(BS, m // rows),
        in_specs=[pl.BlockSpec((1, rows, 3), lambda b, r: (b, r, 0)),
                  pl.BlockSpec((1, 3, n), lambda b, r: (b, 0, 0))],
        out_specs=pl.BlockSpec((1, rows, KNN), lambda b, r: (b, r, 0)),
        out_shape=jax.ShapeDtypeStruct((BS, m, KNN), jnp.int32),
    )(cxyz, xyzt)


def _fps_body(x_ref, y_ref, z_ref, out_ref, *, n, m, s, sm):
    X, Y, Z = x_ref[...], y_ref[...], z_ref[...]
    iota_p = (lax.broadcasted_iota(jnp.int32, (BS, s, 128), 1) * 128
              + lax.broadcasted_iota(jnp.int32, (BS, s, 128), 2))
    iota_o = (lax.broadcasted_iota(jnp.int32, (BS, sm, 128), 1) * 128
              + lax.broadcasted_iota(jnp.int32, (BS, sm, 128), 2))

    def body(i, state):
        dists, last, acc = state
        sel = iota_p == last
        zero = jnp.float32(0.0)
        lx = jnp.sum(jnp.where(sel, X, zero), axis=(1, 2), keepdims=True)
        ly = jnp.sum(jnp.where(sel, Y, zero), axis=(1, 2), keepdims=True)
        lz = jnp.sum(jnp.where(sel, Z, zero), axis=(1, 2), keepdims=True)
        dx, dy, dz = X - lx, Y - ly, Z - lz
        d = dx * dx + dy * dy + dz * dz
        dists = jnp.minimum(dists, d)
        mx = jnp.max(dists, axis=(1, 2), keepdims=True)
        nxt = jnp.min(jnp.where(dists == mx, iota_p, n), axis=(1, 2),
                      keepdims=True)
        acc = jnp.where(iota_o == i, nxt, acc)
        return dists, nxt, acc

    init = (jnp.full((BS, s, 128), 1e10, jnp.float32),
            jnp.zeros((BS, 1, 1), jnp.int32),
            jnp.zeros((BS, sm, 128), jnp.int32))
    _, _, acc = lax.fori_loop(1, m, body, init)
    boffs = lax.broadcasted_iota(jnp.int32, (BS, sm, 128), 0) * n
    out_ref[...] = acc + boffs


def _fps(xflat, yflat, zflat, n, m):
    s, sm = n // 128, m // 128
    body = functools.partial(_fps_body, n=n, m=m, s=s, sm=sm)
    out = pl.pallas_call(
        body,
        out_shape=jax.ShapeDtypeStruct((BS, sm, 128), jnp.int32),
    )(xflat.reshape(BS, s, 128), yflat.reshape(BS, s, 128),
      zflat.reshape(BS, s, 128))
    return out.reshape(BS * m)


def _attn_body(gf_ref, gr_ref, cf_ref, cr_ref, *refs, mb, k, co, specs):
    nb = len(specs)
    wrefs, out_ref = refs[:-1], refs[-1]
    gf = gf_ref[...]
    gr = gr_ref[...]
    cf = cf_ref[...]
    cr = cr_ref[...]
    scale = 1.0 / (co ** 0.5)
    feats = []
    for i, (fc, rc, sign) in enumerate(specs):
        wq, bq, wk, wv, bv = (wrefs[i * 5 + j][...] for j in range(5))
        q = (jnp.dot(cf[:, fc[0]:fc[1]], wq,
                     preferred_element_type=jnp.float32) + bq) * scale
        gcat = jnp.concatenate([gf[:, fc[0]:fc[1]], gr[:, rc[0]:rc[1]]],
                               axis=1)
        cg = gcat.shape[1]
        pk = jnp.dot(gcat, wk[:cg], preferred_element_type=jnp.float32)
        pv = jnp.dot(gcat, wv[:cg], preferred_element_type=jnp.float32)
        craw = cr[:, rc[0]:rc[1]]
        nr = rc[1] - rc[0]
        wvr = wv[cg - nr:cg] if sign < 0 else wv[cg:cg + nr]
        cv = bv + sign * jnp.dot(craw, wvr,
                                 preferred_element_type=jnp.float32)
        pk3 = pk.reshape(mb, k, co)
        pv3 = pv.reshape(mb, k, co)
        q3 = q.reshape(mb, 1, co)
        s3 = jnp.sum(q3 * pk3, axis=2, keepdims=True)
        mx = jnp.max(s3, axis=1, keepdims=True)
        e3 = jnp.exp(s3 - mx)
        a3 = e3 / jnp.sum(e3, axis=1, keepdims=True)
        o = jnp.sum(a3 * pv3, axis=1) + cv
        feats.append(o)
    faw, fab = wrefs[nb * 5][...], wrefs[nb * 5 + 1][...]
    ss = [jnp.sum(f * faw, axis=1, keepdims=True) + fab for f in feats]
    mx = jnp.maximum(jnp.maximum(ss[0], ss[1]), jnp.maximum(ss[2], ss[3]))
    es = [jnp.exp(x - mx) for x in ss]
    z = es[0] + es[1] + es[2] + es[3]
    fused = sum((es[i] / z) * feats[i] for i in range(nb))
    out_ref[...] = jnp.concatenate([f + fused for f in feats], axis=1)


def _attention(gfeat, graw, cfeat, craw, specs, weights, co):
    mt, c = cfeat.shape
    mb = 256
    k = KNN
    flat_w = []
    for (wq, bq, wk, wv, bv) in weights[:-1]:
        flat_w += [wq, bq, wk, wv, bv]
    flat_w += list(weights[-1])
    body = functools.partial(_attn_body, mb=mb, k=k, co=co, specs=specs)
    full = lambda a: pl.BlockSpec(a.shape, lambda i: (0,) * a.ndim)
    return pl.pallas_call(
        body,
        grid=(mt // mb,),
        in_specs=[pl.BlockSpec((mb * k, c), lambda i: (i, 0)),
                  pl.BlockSpec((mb * k, 16), lambda i: (i, 0)),
                  pl.BlockSpec((mb, c), lambda i: (i, 0)),
                  pl.BlockSpec((mb, 16), lambda i: (i, 0))]
        + [full(a) for a in flat_w],
        out_specs=pl.BlockSpec((mb, 4 * co), lambda i: (i, 0)),
        out_shape=jax.ShapeDtypeStruct((mt, 4 * co), jnp.float32),
    )(gfeat, graw, cfeat, craw, *flat_w)


def _final_body(t_ref, r_ref, *refs):
    wrefs, out_ref = refs[:-1], refs[-1]
    t = t_ref[...]
    r = r_ref[...]
    segs = [((0, 128), [(3, 6)]), ((128, 256), [(6, 8), (6, 8)]),
            ((256, 384), [(8, 12), (8, 12)]), ((384, 512), [(0, 3)])]
    gs = []
    for bi, ((f0, f1), extras) in enumerate(segs):
        rows = []
        for b in range(BS):
            tb = t[b * M2:(b + 1) * M2]
            rb = r[b * M2:(b + 1) * M2]
            cols = [tb[:, f0:f1]] + [rb[:, a:bb] for (a, bb) in extras]
            cat = jnp.concatenate(cols, axis=1)
            rows.append(jnp.max(cat, axis=0, keepdims=True))
        g = jnp.concatenate(rows, axis=0)
        w1, b1, w2, b2 = (wrefs[bi * 4 + j][...] for j in range(4))
        h = jnp.maximum(jnp.dot(g, w1, preferred_element_type=jnp.float32)
                        + b1, 0.0)
        gs.append(jnp.maximum(
            jnp.dot(h, w2, preferred_element_type=jnp.float32) + b2, 0.0))
    faw, fab = wrefs[16][...], wrefs[17][...]
    ss = [jnp.sum(f * faw, axis=1, keepdims=True) + fab for f in gs]
    mx = jnp.maximum(jnp.maximum(ss[0], ss[1]), jnp.maximum(ss[2], ss[3]))
    es = [jnp.exp(x - mx) for x in ss]
    z = es[0] + es[1] + es[2] + es[3]
    fused = sum((es[i] / z) * gs[i] for i in range(4))
    gs = [f + fused for f in gs]
    favg = (gs[0] + gs[1] + gs[2] + gs[3]) * 0.25
    hw1, hb1, hw2, hb2 = (wrefs[18 + j][...] for j in range(4))
    h = jnp.maximum(jnp.dot(favg, hw1, preferred_element_type=jnp.float32)
                    + hb1, 0.0)
    out_ref[...] = jnp.dot(h, hw2, preferred_element_type=jnp.float32) + hb2


def _final(table3, craw, wlist):
    return pl.pallas_call(
        _final_body,
        out_shape=jax.ShapeDtypeStruct((BS, 50), jnp.float32),
    )(table3, craw, *wlist)


def _r2(b):
    return b.reshape(1, -1)


def kernel(xyz, pmt, mad, dim, nor, loc, params):
    p = params
    tri = p["tri"]
    names = ["xyz", "pmt", "mad", "dim", "nor", "loc"]
    ins = [3, 5, 3, 1, 3, 3]
    offs = [0, 3, 8, 11, 12, 15]
    w1 = jnp.zeros((18, 96), jnp.float32)
    w2 = jnp.zeros((96, 192), jnp.float32)
    for i, nm in enumerate(names):
        w1 = w1.at[offs[i]:offs[i] + ins[i], 16 * i:16 * (i + 1)].set(
            tri[nm][0]["W"])
        w2 = w2.at[16 * i:16 * (i + 1), 32 * i:32 * (i + 1)].set(tri[nm][1]["W"])
    b1 = jnp.concatenate([tri[nm][0]["b"] for nm in names]).reshape(1, 96)
    b2 = jnp.concatenate([tri[nm][1]["b"] for nm in names]).reshape(1, 192)
    x18 = jnp.concatenate(
        [xyz, pmt, mad, dim[..., None], nor, loc], axis=-1).reshape(-1, 18)
    t1f, t1r = _tri_table(x18, w1, b1, w2, b2,
                          tri["fea"][0]["W"], _r2(tri["fea"][0]["b"]),
                          tri["fea"][1]["W"], _r2(tri["fea"][1]["b"]))

    def attw(ap):
        return (ap["q"]["W"], _r2(ap["q"]["b"]), ap["k"]["W"], ap["v"]["W"],
                _r2(ap["v"]["b"]))

    fps1 = _fps(xyz[:, :, 0], xyz[:, :, 1], xyz[:, :, 2], N1, M1)
    c1f = _gather_rows(t1f, fps1, True)
    c1r = _gather_rows(t1r, fps1, False)
    xyzt = jnp.transpose(xyz, (0, 2, 1))
    cxyz1 = c1r[:, 0:3].reshape(BS, M1, 3)
    nbr1 = _knn(cxyz1, xyzt, N1, M1).reshape(-1)
    g1f = _gather_rows(t1f, nbr1, True)
    g1r = _gather_rows(t1r, nbr1, False)

    specs1 = [((0, 32), (3, 6), -1.0), ((32, 64), (6, 8), 1.0),
              ((64, 96), (8, 12), 1.0), ((96, 128), (0, 3), -1.0)]
    w_1 = [attw(p["ssa1"]["attn_mad"]), attw(p["ssa1"]["attn_adj"]),
           attw(p["ssa1"]["attn_pt"]), attw(p["ssa1"]["attn_cst"]),
           (p["fa1"]["W"].reshape(1, 64), p["fa1"]["b"].reshape(1, 1))]
    t2f = _attention(g1f, g1r, c1f, c1r, specs1, w_1, 64)

    cx = c1r[:, 0].reshape(BS, M1)
    cy = c1r[:, 1].reshape(BS, M1)
    cz = c1r[:, 2].reshape(BS, M1)
    fps2 = _fps(cx, cy, cz, M1, M2)
    c2f = _gather_rows(t2f, fps2, True)
    c2r = _gather_rows(c1r, fps2, False)
    cxyzt = jnp.transpose(cxyz1, (0, 2, 1))
    cxyz2 = c2r[:, 0:3].reshape(BS, M2, 3)
    nbr2 = _knn(cxyz2, cxyzt, M1, M2).reshape(-1)
    g2f = _gather_rows(t2f, nbr2, True)
    g2r = _gather_rows(c1r, nbr2, False)

    specs2 = [((0, 64), (3, 6), -1.0), ((64, 128), (6, 8), 1.0),
              ((128, 192), (8, 12), 1.0), ((192, 256), (0, 3), -1.0)]
    w_2 = [attw(p["ssa2"]["attn_mad"]), attw(p["ssa2"]["attn_adj"]),
           attw(p["ssa2"]["attn_pt"]), attw(p["ssa2"]["attn_cst"]),
           (p["fa2"]["W"].reshape(1, 128), p["fa2"]["b"].reshape(1, 1))]
    t3f = _attention(g2f, g2r, c2f, c2r, specs2, w_2, 128)

    wlist = []
    for nm in ["mlp_mad", "mlp_adj", "mlp_pt", "mlp_cst"]:
        ml = p["ssa3"][nm]
        wlist += [ml[0]["W"], _r2(ml[0]["b"]), ml[1]["W"], _r2(ml[1]["b"])]
    wlist += [p["fa3"]["W"].reshape(1, 256), p["fa3"]["b"].reshape(1, 1)]
    wlist += [p["head"][0]["W"], _r2(p["head"][0]["b"]),
              p["head"][1]["W"], _r2(p["head"][1]["b"])]
    return _final(t3f, c2r, wlist)
